# Initial kernel scaffold; baseline (speedup 1.0000x reference)
import jax
import jax.numpy as jnp
from jax.experimental import pallas as pl


def kernel(pred_logits, pred_boxes, tgt_labels, tgt_boxes):
    raise NotImplementedError("write your pallas kernel here")


# speedup vs baseline: 11.4488x; 11.4488x over previous
import jax
import jax.numpy as jnp
from jax.experimental import pallas as pl
from jax.experimental.pallas import tpu as pltpu

COST_CLASS, COST_BBOX, COST_GIOU = 1.0, 5.0, 2.0
NUM_FRAMES, Q, T, C = 36, 100, 20, 92
NEG = -1e30


def _body(lab_ref, logits_ref, pb_ref, tbt_ref, cost_ref, pi_ref, ti_ref):
    logits = logits_ref[0]
    m = jnp.max(logits, axis=-1, keepdims=True)
    e = jnp.exp(logits - m)
    s = jnp.sum(e, axis=-1, keepdims=True)
    prob = e / s

    lab = lab_ref[0]
    iota_c = jax.lax.broadcasted_iota(jnp.int32, (C, T), 0)
    onehot = jnp.where(iota_c == lab, 1.0, 0.0)
    cost_class = -jax.lax.dot(prob, onehot,
                              precision=jax.lax.Precision.HIGHEST)

    pb = pb_ref[0]
    tbt = tbt_ref[0]
    p_cx, p_cy = pb[:, 0:1], pb[:, 1:2]
    p_w, p_h = pb[:, 2:3], pb[:, 3:4]
    t_cx, t_cy = tbt[0:1, :], tbt[1:2, :]
    t_w, t_h = tbt[2:3, :], tbt[3:4, :]

    cost_bbox = (((jnp.abs(p_cx - t_cx) + jnp.abs(p_cy - t_cy))
                  + jnp.abs(p_w - t_w)) + jnp.abs(p_h - t_h))

    p_x0, p_y0 = p_cx - 0.5 * p_w, p_cy - 0.5 * p_h
    p_x1, p_y1 = p_cx + 0.5 * p_w, p_cy + 0.5 * p_h
    t_x0, t_y0 = t_cx - 0.5 * t_w, t_cy - 0.5 * t_h
    t_x1, t_y1 = t_cx + 0.5 * t_w, t_cy + 0.5 * t_h

    area_p = (p_x1 - p_x0) * (p_y1 - p_y0)
    area_t = (t_x1 - t_x0) * (t_y1 - t_y0)
    iw = jnp.clip(jnp.minimum(p_x1, t_x1) - jnp.maximum(p_x0, t_x0), 0.0)
    ih = jnp.clip(jnp.minimum(p_y1, t_y1) - jnp.maximum(p_y0, t_y0), 0.0)
    inter = iw * ih
    union = area_p + area_t - inter
    iou = inter / union
    ew = jnp.clip(jnp.maximum(p_x1, t_x1) - jnp.minimum(p_x0, t_x0), 0.0)
    eh = jnp.clip(jnp.maximum(p_y1, t_y1) - jnp.minimum(p_y0, t_y0), 0.0)
    area_e = ew * eh
    giou = iou - (area_e - union) / area_e

    cost = (COST_CLASS * cost_class + COST_BBOX * cost_bbox) - COST_GIOU * giou
    cost_ref[0] = cost

    benefit = -jnp.transpose(cost)
    smax = jnp.max(benefit, axis=(0, 1), keepdims=True)
    smin = jnp.min(benefit, axis=(0, 1), keepdims=True)
    eps = (smax - smin + 1e-6) / 1000.0

    iota_q = jax.lax.broadcasted_iota(jnp.int32, (T, Q), 1)
    iota_t = jax.lax.broadcasted_iota(jnp.int32, (T, Q), 0)
    imax = jnp.int32(2147483647)

    def cond(state):
        _, _, obj_of, it = state
        return jnp.logical_and(jnp.any(obj_of < 0), it < 20000)

    def body(state):
        price, owner, obj_of, it = state
        unassigned = obj_of < 0
        vals = benefit - price
        v1 = jnp.max(vals, axis=1, keepdims=True)
        i1 = jnp.min(jnp.where(vals == v1, iota_q, imax),
                     axis=1, keepdims=True)
        sel = iota_q == i1
        v2 = jnp.max(jnp.where(sel, NEG, vals), axis=1, keepdims=True)
        price_at = jnp.sum(jnp.where(sel, price, 0.0), axis=1, keepdims=True)
        bid = (price_at + (v1 - v2)) + eps
        bid = jnp.where(unassigned, bid, NEG)
        best_bid = jnp.max(jnp.where(sel, bid, NEG), axis=0, keepdims=True)
        bb_at = jnp.sum(jnp.where(sel, best_bid, 0.0), axis=1, keepdims=True)
        win = jnp.logical_and(unassigned, bid >= bb_at)
        winner = jnp.min(jnp.where(jnp.logical_and(sel, win), iota_t, T),
                         axis=0, keepdims=True)
        has_bid = winner < T
        evict = jnp.max(jnp.where(jnp.logical_and(has_bid, owner == iota_t),
                                  1, 0), axis=1, keepdims=True) > 0
        new_q = jnp.min(jnp.where(winner == iota_t, iota_q, imax),
                        axis=1, keepdims=True)
        obj_of = jnp.where(evict, -1, obj_of)
        obj_of = jnp.where(new_q < imax, new_q, obj_of)
        price = jnp.where(has_bid, best_bid, price)
        owner = jnp.where(has_bid, winner, owner)
        return price, owner, obj_of, it + 1

    init = (jnp.full((1, Q), 0.0, jnp.float32), jnp.full((1, Q), -1, jnp.int32),
            jnp.full((T, 1), -1, jnp.int32), jnp.int32(0))
    _, _, obj_of, _ = jax.lax.while_loop(cond, body, init)

    obj_row = jnp.transpose(obj_of)
    iota_tc = jax.lax.broadcasted_iota(jnp.int32, (T, T), 0)
    iota_tr = jax.lax.broadcasted_iota(jnp.int32, (T, T), 1)
    less = obj_row < obj_of
    tie = jnp.logical_and(obj_row == obj_of, iota_tr < iota_tc)
    rank = jnp.sum(jnp.where(jnp.logical_or(less, tie), 1, 0),
                   axis=1, keepdims=True)
    hit = rank == iota_tr
    pi_ref[0] = jnp.sum(jnp.where(hit, obj_of, 0), axis=0, keepdims=True)
    ti_ref[0] = jnp.sum(jnp.where(hit, iota_tc, 0), axis=0, keepdims=True)


def kernel(pred_logits, pred_boxes, tgt_labels, tgt_boxes):
    b = pred_logits.shape[0]
    n = b * NUM_FRAMES
    logits3 = pred_logits.reshape(n, Q, C)
    pb3 = pred_boxes.reshape(n, Q, 4)
    lab3 = jnp.transpose(tgt_labels, (1, 0, 2)).reshape(n, 1, T)
    tbt3 = jnp.transpose(tgt_boxes, (1, 0, 3, 2)).reshape(n, 4, T)

    cost, pred_idx, tgt_idx = pl.pallas_call(
        _body,
        grid=(n,),
        in_specs=[
            pl.BlockSpec((1, 1, T), lambda i: (i, 0, 0)),
            pl.BlockSpec((1, Q, C), lambda i: (i, 0, 0)),
            pl.BlockSpec((1, Q, 4), lambda i: (i, 0, 0)),
            pl.BlockSpec((1, 4, T), lambda i: (i, 0, 0)),
        ],
        out_specs=[
            pl.BlockSpec((1, Q, T), lambda i: (i, 0, 0)),
            pl.BlockSpec((1, 1, T), lambda i: (i, 0, 0)),
            pl.BlockSpec((1, 1, T), lambda i: (i, 0, 0)),
        ],
        out_shape=[
            jax.ShapeDtypeStruct((n, Q, T), jnp.float32),
            jax.ShapeDtypeStruct((n, 1, T), jnp.int32),
            jax.ShapeDtypeStruct((n, 1, T), jnp.int32),
        ],
        compiler_params=pltpu.CompilerParams(
            dimension_semantics=("parallel",),
        ),
    )(lab3, logits3, pb3, tbt3)

    return (cost.reshape(b, NUM_FRAMES, Q, T),
            pred_idx.reshape(b, NUM_FRAMES, T),
            tgt_idx.reshape(b, NUM_FRAMES, T))


# speedup vs baseline: 30.7700x; 2.6876x over previous
import jax
import jax.numpy as jnp
from jax.experimental import pallas as pl
from jax.experimental.pallas import tpu as pltpu

COST_CLASS, COST_BBOX, COST_GIOU = 1.0, 5.0, 2.0
NUM_FRAMES, Q, T, C = 36, 100, 20, 92
G = 8
NEG = -1e30


def _body(lab_ref, logits_ref, pb_ref, tbt_ref, cost_ref, pi_ref, ti_ref):
    logits = logits_ref[...]
    m = jnp.max(logits, axis=2, keepdims=True)
    e = jnp.exp(logits - m)
    s = jnp.sum(e, axis=2, keepdims=True)
    prob = e / s

    lab = lab_ref[...]
    iota_c = jax.lax.broadcasted_iota(jnp.int32, (G, C, T), 1)
    onehot = jnp.where(iota_c == lab, 1.0, 0.0)
    cost_class = -jax.lax.dot_general(
        prob, onehot, (((2,), (1,)), ((0,), (0,))),
        precision=jax.lax.Precision.HIGHEST)

    pb = pb_ref[...]
    tbt = tbt_ref[...]
    p_cx, p_cy = pb[:, :, 0:1], pb[:, :, 1:2]
    p_w, p_h = pb[:, :, 2:3], pb[:, :, 3:4]
    t_cx, t_cy = tbt[:, 0:1, :], tbt[:, 1:2, :]
    t_w, t_h = tbt[:, 2:3, :], tbt[:, 3:4, :]

    cost_bbox = (((jnp.abs(p_cx - t_cx) + jnp.abs(p_cy - t_cy))
                  + jnp.abs(p_w - t_w)) + jnp.abs(p_h - t_h))

    p_x0, p_y0 = p_cx - 0.5 * p_w, p_cy - 0.5 * p_h
    p_x1, p_y1 = p_cx + 0.5 * p_w, p_cy + 0.5 * p_h
    t_x0, t_y0 = t_cx - 0.5 * t_w, t_cy - 0.5 * t_h
    t_x1, t_y1 = t_cx + 0.5 * t_w, t_cy + 0.5 * t_h

    area_p = (p_x1 - p_x0) * (p_y1 - p_y0)
    area_t = (t_x1 - t_x0) * (t_y1 - t_y0)
    iw = jnp.clip(jnp.minimum(p_x1, t_x1) - jnp.maximum(p_x0, t_x0), 0.0)
    ih = jnp.clip(jnp.minimum(p_y1, t_y1) - jnp.maximum(p_y0, t_y0), 0.0)
    inter = iw * ih
    union = area_p + area_t - inter
    iou = inter / union
    ew = jnp.clip(jnp.maximum(p_x1, t_x1) - jnp.minimum(p_x0, t_x0), 0.0)
    eh = jnp.clip(jnp.maximum(p_y1, t_y1) - jnp.minimum(p_y0, t_y0), 0.0)
    area_e = ew * eh
    giou = iou - (area_e - union) / area_e

    cost = (COST_CLASS * cost_class + COST_BBOX * cost_bbox) - COST_GIOU * giou
    cost_ref[...] = cost

    benefit = -jnp.transpose(cost, (0, 2, 1))
    smax = jnp.max(benefit, axis=(1, 2), keepdims=True)
    smin = jnp.min(benefit, axis=(1, 2), keepdims=True)
    eps = (smax - smin + 1e-6) / 1000.0

    iota_q = jax.lax.broadcasted_iota(jnp.int32, (G, T, Q), 2)
    iota_t = jax.lax.broadcasted_iota(jnp.int32, (G, T, Q), 1)
    imax = jnp.int32(2147483647)

    def cond(state):
        _, _, obj_of, it = state
        return jnp.logical_and(jnp.any(obj_of < 0), it < 20000)

    def body(state):
        price, owner, obj_of, it = state
        unassigned = obj_of < 0
        vals = benefit - price
        v1 = jnp.max(vals, axis=2, keepdims=True)
        i1 = jnp.min(jnp.where(vals == v1, iota_q, imax),
                     axis=2, keepdims=True)
        sel = iota_q == i1
        v2 = jnp.max(jnp.where(sel, NEG, vals), axis=2, keepdims=True)
        price_at = jnp.sum(jnp.where(sel, price, 0.0), axis=2, keepdims=True)
        bid = (price_at + (v1 - v2)) + eps
        bid = jnp.where(unassigned, bid, NEG)
        best_bid = jnp.max(jnp.where(sel, bid, NEG),
                           axis=1, keepdims=True)
        bb_at = jnp.sum(jnp.where(sel, best_bid, 0.0), axis=2, keepdims=True)
        win = jnp.logical_and(unassigned, bid >= bb_at)
        winner = jnp.min(jnp.where(jnp.logical_and(sel, win), iota_t, T),
                         axis=1, keepdims=True)
        has_bid = winner < T
        evict = jnp.max(jnp.where(jnp.logical_and(has_bid, owner == iota_t),
                                  1, 0), axis=2, keepdims=True) > 0
        new_q = jnp.min(jnp.where(winner == iota_t, iota_q, imax),
                        axis=2, keepdims=True)
        obj_of = jnp.where(evict, -1, obj_of)
        obj_of = jnp.where(new_q < imax, new_q, obj_of)
        price = jnp.where(has_bid, best_bid, price)
        owner = jnp.where(has_bid, winner, owner)
        return price, owner, obj_of, it + 1

    init = (jnp.full((G, 1, Q), 0.0, jnp.float32),
            jnp.full((G, 1, Q), -1, jnp.int32),
            jnp.full((G, T, 1), -1, jnp.int32), jnp.int32(0))
    _, _, obj_of, _ = jax.lax.while_loop(cond, body, init)

    obj_row = jnp.transpose(obj_of, (0, 2, 1))
    iota_tc = jax.lax.broadcasted_iota(jnp.int32, (G, T, T), 1)
    iota_tr = jax.lax.broadcasted_iota(jnp.int32, (G, T, T), 2)
    less = obj_row < obj_of
    tie = jnp.logical_and(obj_row == obj_of, iota_tr < iota_tc)
    rank = jnp.sum(jnp.where(jnp.logical_or(less, tie), 1, 0),
                   axis=2, keepdims=True)
    hit = rank == iota_tr
    pi_ref[...] = jnp.sum(jnp.where(hit, obj_of, 0), axis=1, keepdims=True)
    ti_ref[...] = jnp.sum(jnp.where(hit, iota_tc, 0), axis=1, keepdims=True)


def kernel(pred_logits, pred_boxes, tgt_labels, tgt_boxes):
    b = pred_logits.shape[0]
    n = b * NUM_FRAMES
    logits3 = pred_logits.reshape(n, Q, C)
    pb3 = pred_boxes.reshape(n, Q, 4)
    lab3 = jnp.transpose(tgt_labels, (1, 0, 2)).reshape(n, 1, T)
    tbt3 = jnp.transpose(tgt_boxes, (1, 0, 3, 2)).reshape(n, 4, T)

    cost, pred_idx, tgt_idx = pl.pallas_call(
        _body,
        grid=(n // G,),
        in_specs=[
            pl.BlockSpec((G, 1, T), lambda i: (i, 0, 0)),
            pl.BlockSpec((G, Q, C), lambda i: (i, 0, 0)),
            pl.BlockSpec((G, Q, 4), lambda i: (i, 0, 0)),
            pl.BlockSpec((G, 4, T), lambda i: (i, 0, 0)),
        ],
        out_specs=[
            pl.BlockSpec((G, Q, T), lambda i: (i, 0, 0)),
            pl.BlockSpec((G, 1, T), lambda i: (i, 0, 0)),
            pl.BlockSpec((G, 1, T), lambda i: (i, 0, 0)),
        ],
        out_shape=[
            jax.ShapeDtypeStruct((n, Q, T), jnp.float32),
            jax.ShapeDtypeStruct((n, 1, T), jnp.int32),
            jax.ShapeDtypeStruct((n, 1, T), jnp.int32),
        ],
        compiler_params=pltpu.CompilerParams(
            dimension_semantics=("parallel",),
        ),
    )(lab3, logits3, pb3, tbt3)

    return (cost.reshape(b, NUM_FRAMES, Q, T),
            pred_idx.reshape(b, NUM_FRAMES, T),
            tgt_idx.reshape(b, NUM_FRAMES, T))


# speedup vs baseline: 34.9366x; 1.1354x over previous
import jax
import jax.numpy as jnp
from jax.experimental import pallas as pl
from jax.experimental.pallas import tpu as pltpu

COST_CLASS, COST_BBOX, COST_GIOU = 1.0, 5.0, 2.0
NUM_FRAMES, Q, T, C = 36, 100, 20, 92
G = 8
NEG = -1e30


def _body(lab_ref, logits_ref, pbt_ref, tb_ref, cost_ref, pi_ref, ti_ref):
    logits = logits_ref[...]
    m = jnp.max(logits, axis=2, keepdims=True)
    e = jnp.exp(logits - m)
    s = jnp.sum(e, axis=2, keepdims=True)
    prob = e / s

    lab = lab_ref[...]
    iota_c = jax.lax.broadcasted_iota(jnp.int32, (G, C, T), 1)
    onehot = jnp.where(iota_c == lab, 1.0, 0.0)
    gathered = jax.lax.dot_general(
        onehot, prob, (((1,), (2,)), ((0,), (0,))),
        precision=jax.lax.Precision.HIGHEST)
    cost_class = -gathered.astype(jnp.bfloat16).astype(jnp.float32)

    pbt = pbt_ref[...]
    tb = tb_ref[...]
    p_cx, p_cy = pbt[:, 0:1, :], pbt[:, 1:2, :]
    p_w, p_h = pbt[:, 2:3, :], pbt[:, 3:4, :]
    t_cx, t_cy = tb[:, :, 0:1], tb[:, :, 1:2]
    t_w, t_h = tb[:, :, 2:3], tb[:, :, 3:4]

    cost_bbox = (((jnp.abs(p_cx - t_cx) + jnp.abs(p_cy - t_cy))
                  + jnp.abs(p_w - t_w)) + jnp.abs(p_h - t_h))

    p_x0, p_y0 = p_cx - 0.5 * p_w, p_cy - 0.5 * p_h
    p_x1, p_y1 = p_cx + 0.5 * p_w, p_cy + 0.5 * p_h
    t_x0, t_y0 = t_cx - 0.5 * t_w, t_cy - 0.5 * t_h
    t_x1, t_y1 = t_cx + 0.5 * t_w, t_cy + 0.5 * t_h

    area_p = (p_x1 - p_x0) * (p_y1 - p_y0)
    area_t = (t_x1 - t_x0) * (t_y1 - t_y0)
    iw = jnp.clip(jnp.minimum(p_x1, t_x1) - jnp.maximum(p_x0, t_x0), 0.0)
    ih = jnp.clip(jnp.minimum(p_y1, t_y1) - jnp.maximum(p_y0, t_y0), 0.0)
    inter = iw * ih
    union = area_p + area_t - inter
    iou = inter / union
    ew = jnp.clip(jnp.maximum(p_x1, t_x1) - jnp.minimum(p_x0, t_x0), 0.0)
    eh = jnp.clip(jnp.maximum(p_y1, t_y1) - jnp.minimum(p_y0, t_y0), 0.0)
    area_e = ew * eh
    giou = iou - (area_e - union) / area_e

    cost_t = (COST_CLASS * cost_class + COST_BBOX * cost_bbox) - COST_GIOU * giou
    cost_ref[...] = jnp.transpose(cost_t, (0, 2, 1))

    benefit = -cost_t
    smax = jnp.max(benefit, axis=(1, 2), keepdims=True)
    smin = jnp.min(benefit, axis=(1, 2), keepdims=True)
    eps = (smax - smin + 1e-6) / 1000.0

    iota_q = jax.lax.broadcasted_iota(jnp.int32, (G, T, Q), 2)
    iota_t = jax.lax.broadcasted_iota(jnp.int32, (G, T, Q), 1)
    imax = jnp.int32(2147483647)

    def cond(state):
        _, _, obj_of, it = state
        return jnp.logical_and(jnp.any(obj_of < 0), it < 20000)

    def body(state):
        price, owner, obj_of, it = state
        unassigned = obj_of < 0
        vals = benefit - price
        v1 = jnp.max(vals, axis=2, keepdims=True)
        i1 = jnp.min(jnp.where(vals == v1, iota_q, imax),
                     axis=2, keepdims=True)
        sel = iota_q == i1
        v2 = jnp.max(jnp.where(sel, NEG, vals), axis=2, keepdims=True)
        price_at = jnp.sum(jnp.where(sel, price, 0.0), axis=2, keepdims=True)
        bid = (price_at + (v1 - v2)) + eps
        bid = jnp.where(unassigned, bid, NEG)
        best_bid = jnp.max(jnp.where(sel, bid, NEG),
                           axis=1, keepdims=True)
        bb_at = jnp.sum(jnp.where(sel, best_bid, 0.0), axis=2, keepdims=True)
        win = jnp.logical_and(unassigned, bid >= bb_at)
        winner = jnp.min(jnp.where(jnp.logical_and(sel, win), iota_t, T),
                         axis=1, keepdims=True)
        has_bid = winner < T
        evict = jnp.max(jnp.where(jnp.logical_and(has_bid, owner == iota_t),
                                  1, 0), axis=2, keepdims=True) > 0
        new_q = jnp.min(jnp.where(winner == iota_t, iota_q, imax),
                        axis=2, keepdims=True)
        obj_of = jnp.where(evict, -1, obj_of)
        obj_of = jnp.where(new_q < imax, new_q, obj_of)
        price = jnp.where(has_bid, best_bid, price)
        owner = jnp.where(has_bid, winner, owner)
        return price, owner, obj_of, it + 1

    init = (jnp.full((G, 1, Q), 0.0, jnp.float32),
            jnp.full((G, 1, Q), -1, jnp.int32),
            jnp.full((G, T, 1), -1, jnp.int32), jnp.int32(0))
    _, _, obj_of, _ = jax.lax.while_loop(cond, body, init)

    obj_row = jnp.transpose(obj_of, (0, 2, 1))
    iota_tc = jax.lax.broadcasted_iota(jnp.int32, (G, T, T), 1)
    iota_tr = jax.lax.broadcasted_iota(jnp.int32, (G, T, T), 2)
    less = obj_row < obj_of
    tie = jnp.logical_and(obj_row == obj_of, iota_tr < iota_tc)
    rank = jnp.sum(jnp.where(jnp.logical_or(less, tie), 1, 0),
                   axis=2, keepdims=True)
    hit = rank == iota_tr
    pi_ref[...] = jnp.sum(jnp.where(hit, obj_of, 0), axis=1, keepdims=True)
    ti_ref[...] = jnp.sum(jnp.where(hit, iota_tc, 0), axis=1, keepdims=True)


def kernel(pred_logits, pred_boxes, tgt_labels, tgt_boxes):
    b = pred_logits.shape[0]
    n = b * NUM_FRAMES
    logits3 = pred_logits.reshape(n, Q, C)
    pbt3 = jnp.transpose(pred_boxes.reshape(n, Q, 4), (0, 2, 1))
    lab3 = jnp.transpose(tgt_labels, (1, 0, 2)).reshape(n, 1, T)
    tb3 = jnp.transpose(tgt_boxes, (1, 0, 2, 3)).reshape(n, T, 4)

    cost, pred_idx, tgt_idx = pl.pallas_call(
        _body,
        grid=(n // G,),
        in_specs=[
            pl.BlockSpec((G, 1, T), lambda i: (i, 0, 0)),
            pl.BlockSpec((G, Q, C), lambda i: (i, 0, 0)),
            pl.BlockSpec((G, 4, Q), lambda i: (i, 0, 0)),
            pl.BlockSpec((G, T, 4), lambda i: (i, 0, 0)),
        ],
        out_specs=[
            pl.BlockSpec((G, Q, T), lambda i: (i, 0, 0)),
            pl.BlockSpec((G, 1, T), lambda i: (i, 0, 0)),
            pl.BlockSpec((G, 1, T), lambda i: (i, 0, 0)),
        ],
        out_shape=[
            jax.ShapeDtypeStruct((n, Q, T), jnp.float32),
            jax.ShapeDtypeStruct((n, 1, T), jnp.int32),
            jax.ShapeDtypeStruct((n, 1, T), jnp.int32),
        ],
        compiler_params=pltpu.CompilerParams(
            dimension_semantics=("parallel",),
        ),
    )(lab3, logits3, pbt3, tb3)

    return (cost.reshape(b, NUM_FRAMES, Q, T),
            pred_idx.reshape(b, NUM_FRAMES, T),
            tgt_idx.reshape(b, NUM_FRAMES, T))


# speedup vs baseline: 38.3885x; 1.0988x over previous
import jax
import jax.numpy as jnp
from jax.experimental import pallas as pl
from jax.experimental.pallas import tpu as pltpu

COST_CLASS, COST_BBOX, COST_GIOU = 1.0, 5.0, 2.0
NUM_FRAMES, Q, T, C = 36, 100, 20, 92
G = 8
NEG = -1e30


def _body(lab_ref, logits_ref, pbt_ref, tb_ref, cost_ref, pi_ref, ti_ref):
    logits = logits_ref[...]
    m = jnp.max(logits, axis=2, keepdims=True)
    e = jnp.exp(logits - m)
    s = jnp.sum(e, axis=2, keepdims=True)
    prob = e / s

    lab = lab_ref[...]
    iota_c = jax.lax.broadcasted_iota(jnp.int32, (G, T, C), 2)
    onehot = jnp.where(iota_c == lab, 1.0, 0.0)
    gathered = jax.lax.dot_general(
        onehot, prob, (((2,), (2,)), ((0,), (0,))),
        precision=jax.lax.Precision.HIGHEST)
    cost_class = -gathered.astype(jnp.bfloat16).astype(jnp.float32)

    pbt = pbt_ref[...]
    tb = tb_ref[...]
    p_cx, p_cy = pbt[:, 0:1, :], pbt[:, 1:2, :]
    p_w, p_h = pbt[:, 2:3, :], pbt[:, 3:4, :]
    t_cx, t_cy = tb[:, :, 0:1], tb[:, :, 1:2]
    t_w, t_h = tb[:, :, 2:3], tb[:, :, 3:4]

    cost_bbox = (((jnp.abs(p_cx - t_cx) + jnp.abs(p_cy - t_cy))
                  + jnp.abs(p_w - t_w)) + jnp.abs(p_h - t_h))

    p_x0, p_y0 = p_cx - 0.5 * p_w, p_cy - 0.5 * p_h
    p_x1, p_y1 = p_cx + 0.5 * p_w, p_cy + 0.5 * p_h
    t_x0, t_y0 = t_cx - 0.5 * t_w, t_cy - 0.5 * t_h
    t_x1, t_y1 = t_cx + 0.5 * t_w, t_cy + 0.5 * t_h

    area_p = (p_x1 - p_x0) * (p_y1 - p_y0)
    area_t = (t_x1 - t_x0) * (t_y1 - t_y0)
    iw = jnp.clip(jnp.minimum(p_x1, t_x1) - jnp.maximum(p_x0, t_x0), 0.0)
    ih = jnp.clip(jnp.minimum(p_y1, t_y1) - jnp.maximum(p_y0, t_y0), 0.0)
    inter = iw * ih
    union = area_p + area_t - inter
    iou = inter / union
    ew = jnp.clip(jnp.maximum(p_x1, t_x1) - jnp.minimum(p_x0, t_x0), 0.0)
    eh = jnp.clip(jnp.maximum(p_y1, t_y1) - jnp.minimum(p_y0, t_y0), 0.0)
    area_e = ew * eh
    giou = iou - (area_e - union) / area_e

    cost_t = (COST_CLASS * cost_class + COST_BBOX * cost_bbox) - COST_GIOU * giou
    cost_ref[...] = jnp.transpose(cost_t, (0, 2, 1))

    benefit = -cost_t
    smax = jnp.max(benefit, axis=(1, 2), keepdims=True)
    smin = jnp.min(benefit, axis=(1, 2), keepdims=True)
    eps = (smax - smin + 1e-6) / 1000.0

    iota_q = jax.lax.broadcasted_iota(jnp.int32, (G, T, Q), 2)
    iota_t = jax.lax.broadcasted_iota(jnp.int32, (G, T, Q), 1)
    imax = jnp.int32(2147483647)

    def cond(state):
        _, owner, it = state
        n_assigned = jnp.sum(jnp.where(owner >= 0, 1, 0),
                             axis=(1, 2), keepdims=True)
        return jnp.logical_and(jnp.any(n_assigned < T), it < 20000)

    def body(state):
        price, owner, it = state
        assigned_t = jnp.max(jnp.where(owner == iota_t, 1, 0),
                             axis=2, keepdims=True)
        unassigned = assigned_t == 0
        vals = benefit - price
        v1 = jnp.max(vals, axis=2, keepdims=True)
        i1 = jnp.min(jnp.where(vals == v1, iota_q, imax),
                     axis=2, keepdims=True)
        sel = iota_q == i1
        v2 = jnp.max(jnp.where(sel, NEG, vals), axis=2, keepdims=True)
        price_at = jnp.sum(jnp.where(sel, price, 0.0), axis=2, keepdims=True)
        bid = (price_at + (v1 - v2)) + eps
        bid = jnp.where(unassigned, bid, NEG)
        best_bid = jnp.max(jnp.where(sel, bid, NEG),
                           axis=1, keepdims=True)
        bb_at = jnp.sum(jnp.where(sel, best_bid, 0.0), axis=2, keepdims=True)
        win = jnp.logical_and(unassigned, bid >= bb_at)
        winner = jnp.min(jnp.where(jnp.logical_and(sel, win), iota_t, T),
                         axis=1, keepdims=True)
        has_bid = winner < T
        price = jnp.where(has_bid, best_bid, price)
        owner = jnp.where(has_bid, winner, owner)
        return price, owner, it + 1

    init = (jnp.full((G, 1, Q), 0.0, jnp.float32),
            jnp.full((G, 1, Q), -1, jnp.int32), jnp.int32(0))
    _, owner, _ = jax.lax.while_loop(cond, body, init)

    obj_of = jnp.min(jnp.where(owner == iota_t, iota_q, imax),
                     axis=2, keepdims=True)
    obj_of = jnp.where(obj_of == imax, -1, obj_of)

    obj_row = jnp.transpose(obj_of, (0, 2, 1))
    iota_tc = jax.lax.broadcasted_iota(jnp.int32, (G, T, T), 1)
    iota_tr = jax.lax.broadcasted_iota(jnp.int32, (G, T, T), 2)
    less = obj_row < obj_of
    tie = jnp.logical_and(obj_row == obj_of, iota_tr < iota_tc)
    rank = jnp.sum(jnp.where(jnp.logical_or(less, tie), 1, 0),
                   axis=2, keepdims=True)
    hit = rank == iota_tr
    pi_ref[...] = jnp.sum(jnp.where(hit, obj_of, 0), axis=1, keepdims=True)
    ti_ref[...] = jnp.sum(jnp.where(hit, iota_tc, 0), axis=1, keepdims=True)


def kernel(pred_logits, pred_boxes, tgt_labels, tgt_boxes):
    b = pred_logits.shape[0]
    n = b * NUM_FRAMES
    logits3 = pred_logits.reshape(n, Q, C)
    pbt3 = jnp.transpose(pred_boxes.reshape(n, Q, 4), (0, 2, 1))
    lab3 = jnp.transpose(tgt_labels, (1, 0, 2)).reshape(n, T, 1)
    tb3 = jnp.transpose(tgt_boxes, (1, 0, 2, 3)).reshape(n, T, 4)

    cost, pred_idx, tgt_idx = pl.pallas_call(
        _body,
        grid=(n // G,),
        in_specs=[
            pl.BlockSpec((G, T, 1), lambda i: (i, 0, 0)),
            pl.BlockSpec((G, Q, C), lambda i: (i, 0, 0)),
            pl.BlockSpec((G, 4, Q), lambda i: (i, 0, 0)),
            pl.BlockSpec((G, T, 4), lambda i: (i, 0, 0)),
        ],
        out_specs=[
            pl.BlockSpec((G, Q, T), lambda i: (i, 0, 0)),
            pl.BlockSpec((G, 1, T), lambda i: (i, 0, 0)),
            pl.BlockSpec((G, 1, T), lambda i: (i, 0, 0)),
        ],
        out_shape=[
            jax.ShapeDtypeStruct((n, Q, T), jnp.float32),
            jax.ShapeDtypeStruct((n, 1, T), jnp.int32),
            jax.ShapeDtypeStruct((n, 1, T), jnp.int32),
        ],
        compiler_params=pltpu.CompilerParams(
            dimension_semantics=("parallel",),
        ),
    )(lab3, logits3, pbt3, tb3)

    return (cost.reshape(b, NUM_FRAMES, Q, T),
            pred_idx.reshape(b, NUM_FRAMES, T),
            tgt_idx.reshape(b, NUM_FRAMES, T))
